# baseline (speedup 1.0000x reference)
import functools

import jax
import jax.numpy as jnp
from jax import lax
from jax.experimental import pallas as pl
from jax.experimental.pallas import tpu as pltpu
from jax.experimental.pallas import tpu_sc as plsc

_INT_MIN = -2147483648
_LANES = 16


def _keys(s):
    b = lax.bitcast_convert_type(s, jnp.int32)
    return jnp.where(b >= 0, b, jnp.int32(_INT_MIN) - b)


def _make_sc_select(nc, npad, k, idx_bits):
    inv_max = (1 << idx_bits) - 1
    nchunk = npad // _LANES
    mesh = plsc.VectorSubcoreMesh(core_axis_name="c", subcore_axis_name="s")

    ku_levels = [(27, 31), (22, 31), (17, 31), (12, 31), (7, 31), (2, 31),
                 (0, 3)]
    idx_levels = [(12, 31), (7, 31), (2, 31), (0, 3)]

    @functools.partial(
        pl.kernel,
        mesh=mesh,
        out_type=jax.ShapeDtypeStruct((nc, _LANES), jnp.int32),
        scratch_types=[
            pltpu.VMEM((npad,), jnp.int32),
            pltpu.VMEM((32 * _LANES,), jnp.int32),
            pltpu.VMEM((_LANES,), jnp.int32),
        ],
        compiler_params=pltpu.CompilerParams(needs_layout_passes=False),
    )
    def sc_select(keys_hbm, out_hbm, data_v, hist_v, row_v):
        wid = lax.axis_index("s") * 2 + lax.axis_index("c")

        @pl.when(wid < nc)
        def _():
            lane = lax.iota(jnp.int32, _LANES)
            ones = jnp.ones((_LANES,), jnp.int32)
            int_min = jnp.int32(_INT_MIN)

            pltpu.sync_copy(keys_hbm.at[wid], data_v)

            def zero_hist(nbins):
                for d in range(nbins):
                    hist_v[pl.ds(d * _LANES, _LANES)] = jnp.zeros(
                        (_LANES,), jnp.int32)

            def scan(digit_of, pred_of):
                def body(i, _):
                    off = i * (2 * _LANES)
                    for h in range(2):
                        o = off + h * _LANES
                        ku = data_v[pl.ds(o, _LANES)]
                        inv = jnp.int32(inv_max) - (o + lane)
                        dg = digit_of(ku, inv)
                        pr = pred_of(ku, inv)
                        if pr is None:
                            pr = ones > jnp.int32(0)
                        plsc.addupdate_scatter(
                            hist_v, [dg * _LANES + lane], ones, mask=pr)
                    return 0

                lax.fori_loop(0, nchunk // 2, body, 0)

            def pick(nbins, rank):
                acc = jnp.int32(0)
                dsel = jnp.int32(0)
                rnew = rank
                hsel = jnp.int32(0)
                for d in reversed(range(nbins)):
                    h = jnp.sum(hist_v[pl.ds(d * _LANES, _LANES)])
                    hit = (acc < rank) & (acc + h >= rank)
                    dsel = jnp.where(hit, jnp.int32(d), dsel)
                    rnew = jnp.where(hit, rank - acc, rnew)
                    hsel = jnp.where(hit, h, hsel)
                    acc = acc + h
                return dsel, rnew, hsel

            p = jnp.int32(0)
            rank = jnp.int32(k)
            n_eq = jnp.int32(0)
            for li, (dsh, dmask) in enumerate(ku_levels):
                nbins = dmask + 1
                zero_hist(nbins)
                psh = 32 - 5 * li
                pcur = p

                def dof(ku, inv, dsh=dsh, dmask=dmask):
                    return lax.shift_right_logical(ku, dsh) & dmask

                if li == 0:
                    def pof(ku, inv):
                        return None
                else:
                    def pof(ku, inv, psh=psh, pcur=pcur):
                        return lax.shift_right_logical(ku, psh) == pcur

                scan(dof, pof)
                d, rank, hsel = pick(nbins, rank)
                p = lax.shift_left(p, (nbins - 1).bit_length()) | d
                n_eq = hsel
            thr_ku = p

            def tie_path(rank):
                pv = jnp.int32(0)
                rk = rank
                for li, (dsh, dmask) in enumerate(idx_levels):
                    nbins = dmask + 1
                    zero_hist(nbins)
                    psh = idx_bits - 5 * li
                    pvcur = pv

                    def dof(ku, inv, dsh=dsh, dmask=dmask):
                        return lax.shift_right_logical(inv, dsh) & dmask

                    if li == 0:
                        def pof(ku, inv):
                            return ku == thr_ku
                    else:
                        def pof(ku, inv, psh=psh, pvcur=pvcur):
                            return (ku == thr_ku) & (
                                lax.shift_right_logical(inv, psh) == pvcur)

                    scan(dof, pof)
                    d, rk, _ = pick(nbins, rk)
                    pv = lax.shift_left(pv, (nbins - 1).bit_length()) | d
                return jnp.int32(inv_max) - pv

            m_cut = lax.cond(rank == n_eq,
                             lambda: jnp.int32(inv_max),
                             lambda: tie_path(rank))

            thr_signed = thr_ku ^ int_min
            row = jnp.where(lane == 0, thr_signed,
                            jnp.where(lane == 1, m_cut, jnp.int32(0)))
            row_v[...] = row
            pltpu.sync_copy(row_v, out_hbm.at[wid])

    return sc_select


def _tc_kernel(scores_blk_ref, feat_ref, thr_ref, scale_ref, out_ref, *,
               nsteps, br):
    step = pl.program_id(0)
    nc = out_ref.shape[0]

    @pl.when(step == 0)
    def _():
        out_ref[...] = jnp.zeros_like(out_ref)

    kblk = _keys(scores_blk_ref[0])
    iblk = lax.broadcasted_iota(jnp.int32, (nc, br), 1) + step * br
    tm = thr_ref[...]
    thr = tm[:, 0:1]
    m = tm[:, 1:2]
    sel = (kblk > thr) | ((kblk == thr) & (iblk <= m))
    w = jnp.where(sel, 1.0, 0.0)
    out_ref[...] += lax.dot_general(
        w, feat_ref[...], (((1,), (0,)), ((), ())),
        preferred_element_type=jnp.float32)

    @pl.when(step == nsteps - 1)
    def _finish():
        out_ref[...] *= scale_ref[0, 0]


def kernel(image_features, image_scores, xi_c):
    n, d = image_features.shape
    nc = image_scores.shape[1]
    k = n // nc
    br = 5000
    nsteps = n // br
    npad = ((n + 127) // 128) * 128
    idx_bits = 17

    scores_t = image_scores.T
    scores_tp = jnp.pad(scores_t, ((0, 0), (0, npad - n)),
                        constant_values=-jnp.inf)
    scores_blocks = scores_t.reshape(nc, nsteps, br).transpose(1, 0, 2)

    topk_mask = (jnp.arange(k) < k * xi_c).astype(image_features.dtype)
    scale = (1.0 / jnp.sum(topk_mask)).astype(jnp.float32).reshape(1, 1)

    keys_u = _keys(scores_tp) ^ jnp.int32(_INT_MIN)
    thr_m = _make_sc_select(nc, npad, k, idx_bits)(keys_u)

    body = functools.partial(_tc_kernel, nsteps=nsteps, br=br)
    return pl.pallas_call(
        body,
        grid=(nsteps,),
        in_specs=[
            pl.BlockSpec((1, nc, br), lambda j: (j, 0, 0)),
            pl.BlockSpec((br, d), lambda j: (j, 0)),
            pl.BlockSpec((nc, _LANES), lambda j: (0, 0)),
            pl.BlockSpec(memory_space=pltpu.SMEM),
        ],
        out_specs=pl.BlockSpec((nc, d), lambda j: (0, 0)),
        out_shape=jax.ShapeDtypeStruct((nc, d), jnp.float32),
        compiler_params=pltpu.CompilerParams(
            dimension_semantics=("arbitrary",)),
    )(scores_blocks, image_features, thr_m, scale)


# speedup vs baseline: 1.0674x; 1.0674x over previous
import functools

import jax
import jax.numpy as jnp
from jax import lax
from jax.experimental import pallas as pl
from jax.experimental.pallas import tpu as pltpu
from jax.experimental.pallas import tpu_sc as plsc

_INT_MIN = -2147483648
_LANES = 16


def _keys(s):
    b = lax.bitcast_convert_type(s, jnp.int32)
    return jnp.where(b >= 0, b, jnp.int32(_INT_MIN) - b)


def _make_sc_select(nc, npad, k, idx_bits):
    inv_max = (1 << idx_bits) - 1
    nchunk = npad // _LANES
    mesh = plsc.VectorSubcoreMesh(core_axis_name="c", subcore_axis_name="s")

    ku_levels = [(27, 31), (22, 31), (17, 31), (12, 31), (7, 31), (2, 31),
                 (0, 3)]
    idx_levels = [(12, 31), (7, 31), (2, 31), (0, 3)]

    @functools.partial(
        pl.kernel,
        mesh=mesh,
        out_type=jax.ShapeDtypeStruct((nc, _LANES), jnp.int32),
        scratch_types=[
            pltpu.VMEM((npad,), jnp.int32),
            pltpu.VMEM((32 * _LANES,), jnp.int32),
            pltpu.VMEM((_LANES,), jnp.int32),
        ],
        compiler_params=pltpu.CompilerParams(needs_layout_passes=False),
    )
    def sc_select(keys_hbm, out_hbm, data_v, hist_v, row_v):
        wid = lax.axis_index("s") * 2 + lax.axis_index("c")

        @pl.when(wid < nc)
        def _():
            lane = lax.iota(jnp.int32, _LANES)
            ones = jnp.ones((_LANES,), jnp.int32)
            int_min = jnp.int32(_INT_MIN)

            pltpu.sync_copy(keys_hbm.at[wid], data_v)

            def zero_hist(nbins):
                for d in range(nbins):
                    hist_v[pl.ds(d * _LANES, _LANES)] = jnp.zeros(
                        (_LANES,), jnp.int32)

            def scan(digit_of, pred_of):
                unroll = 8

                def body(i, _):
                    off = i * (unroll * _LANES)
                    for h in range(unroll):
                        o = off + h * _LANES
                        ku = data_v[pl.ds(o, _LANES)]
                        inv = jnp.int32(inv_max) - (o + lane)
                        dg = digit_of(ku, inv)
                        pr = pred_of(ku, inv)
                        if pr is None:
                            pr = ones > jnp.int32(0)
                        plsc.addupdate_scatter(
                            hist_v, [dg * _LANES + lane], ones, mask=pr)
                    return 0

                lax.fori_loop(0, nchunk // unroll, body, 0)

            def pick(nbins, rank):
                acc = jnp.int32(0)
                dsel = jnp.int32(0)
                rnew = rank
                hsel = jnp.int32(0)
                for d in reversed(range(nbins)):
                    h = jnp.sum(hist_v[pl.ds(d * _LANES, _LANES)])
                    hit = (acc < rank) & (acc + h >= rank)
                    dsel = jnp.where(hit, jnp.int32(d), dsel)
                    rnew = jnp.where(hit, rank - acc, rnew)
                    hsel = jnp.where(hit, h, hsel)
                    acc = acc + h
                return dsel, rnew, hsel

            p = jnp.int32(0)
            rank = jnp.int32(k)
            n_eq = jnp.int32(0)
            for li, (dsh, dmask) in enumerate(ku_levels):
                nbins = dmask + 1
                zero_hist(nbins)
                psh = 32 - 5 * li
                pcur = p

                def dof(ku, inv, dsh=dsh, dmask=dmask):
                    return lax.shift_right_logical(ku, dsh) & dmask

                if li == 0:
                    def pof(ku, inv):
                        return None
                else:
                    def pof(ku, inv, psh=psh, pcur=pcur):
                        return lax.shift_right_logical(ku, psh) == pcur

                scan(dof, pof)
                d, rank, hsel = pick(nbins, rank)
                p = lax.shift_left(p, (nbins - 1).bit_length()) | d
                n_eq = hsel
            thr_ku = p

            def tie_path(rank):
                pv = jnp.int32(0)
                rk = rank
                for li, (dsh, dmask) in enumerate(idx_levels):
                    nbins = dmask + 1
                    zero_hist(nbins)
                    psh = idx_bits - 5 * li
                    pvcur = pv

                    def dof(ku, inv, dsh=dsh, dmask=dmask):
                        return lax.shift_right_logical(inv, dsh) & dmask

                    if li == 0:
                        def pof(ku, inv):
                            return ku == thr_ku
                    else:
                        def pof(ku, inv, psh=psh, pvcur=pvcur):
                            return (ku == thr_ku) & (
                                lax.shift_right_logical(inv, psh) == pvcur)

                    scan(dof, pof)
                    d, rk, _ = pick(nbins, rk)
                    pv = lax.shift_left(pv, (nbins - 1).bit_length()) | d
                return jnp.int32(inv_max) - pv

            m_cut = lax.cond(rank == n_eq,
                             lambda: jnp.int32(inv_max),
                             lambda: tie_path(rank))

            thr_signed = thr_ku ^ int_min
            row = jnp.where(lane == 0, thr_signed,
                            jnp.where(lane == 1, m_cut, jnp.int32(0)))
            row_v[...] = row
            pltpu.sync_copy(row_v, out_hbm.at[wid])

    return sc_select


def _tc_kernel(scores_blk_ref, feat_ref, thr_ref, scale_ref, out_ref, *,
               nsteps, br):
    step = pl.program_id(0)
    nc = out_ref.shape[0]

    @pl.when(step == 0)
    def _():
        out_ref[...] = jnp.zeros_like(out_ref)

    kblk = _keys(scores_blk_ref[0])
    iblk = lax.broadcasted_iota(jnp.int32, (nc, br), 1) + step * br
    tm = thr_ref[...]
    thr = tm[:, 0:1]
    m = tm[:, 1:2]
    sel = (kblk > thr) | ((kblk == thr) & (iblk <= m))
    w = jnp.where(sel, 1.0, 0.0)
    out_ref[...] += lax.dot_general(
        w, feat_ref[...], (((1,), (0,)), ((), ())),
        preferred_element_type=jnp.float32)

    @pl.when(step == nsteps - 1)
    def _finish():
        out_ref[...] *= scale_ref[0, 0]


def kernel(image_features, image_scores, xi_c):
    n, d = image_features.shape
    nc = image_scores.shape[1]
    k = n // nc
    br = 5000
    nsteps = n // br
    npad = ((n + 127) // 128) * 128
    idx_bits = 17

    scores_t = image_scores.T
    scores_tp = jnp.pad(scores_t, ((0, 0), (0, npad - n)),
                        constant_values=-jnp.inf)
    scores_blocks = scores_t.reshape(nc, nsteps, br).transpose(1, 0, 2)

    topk_mask = (jnp.arange(k) < k * xi_c).astype(image_features.dtype)
    scale = (1.0 / jnp.sum(topk_mask)).astype(jnp.float32).reshape(1, 1)

    keys_u = _keys(scores_tp) ^ jnp.int32(_INT_MIN)
    thr_m = _make_sc_select(nc, npad, k, idx_bits)(keys_u)

    body = functools.partial(_tc_kernel, nsteps=nsteps, br=br)
    return pl.pallas_call(
        body,
        grid=(nsteps,),
        in_specs=[
            pl.BlockSpec((1, nc, br), lambda j: (j, 0, 0)),
            pl.BlockSpec((br, d), lambda j: (j, 0)),
            pl.BlockSpec((nc, _LANES), lambda j: (0, 0)),
            pl.BlockSpec(memory_space=pltpu.SMEM),
        ],
        out_specs=pl.BlockSpec((nc, d), lambda j: (0, 0)),
        out_shape=jax.ShapeDtypeStruct((nc, d), jnp.float32),
        compiler_params=pltpu.CompilerParams(
            dimension_semantics=("arbitrary",)),
    )(scores_blocks, image_features, thr_m, scale)


# speedup vs baseline: 5.1320x; 4.8081x over previous
import functools

import jax
import jax.numpy as jnp
from jax.experimental import pallas as pl
from jax.experimental.pallas import tpu as pltpu

_NC = 10

_INT_MIN = -2147483648


def _keys(s):
    b = jax.lax.bitcast_convert_type(s, jnp.int32)
    return jnp.where(b >= 0, b, jnp.int32(_INT_MIN) - b)


def _cc_kernel(scores_full_ref, scores_blk_ref, feat_ref, scale_ref, out_ref,
               keys_ref, thr_ref, m_ref, *, nsteps, br, k, npad, idx_bits):
    step = pl.program_id(0)
    nc = out_ref.shape[0]
    kf = jnp.float32(k)

    @pl.when(step == 0)
    def _selection():
        g = 8
        rows = nc * g
        ll = npad // g
        keys_ref[...] = _keys(scores_full_ref[...])
        out_ref[...] = jnp.zeros_like(out_ref)

        ri = jax.lax.broadcasted_iota(jnp.int32, (rows, rows), 0)
        ci = jax.lax.broadcasted_iota(jnp.int32, (rows, rows), 1)
        g2 = jnp.where(ri // g == ci // g, 1.0, 0.0)

        def gsum(part):
            return jax.lax.dot_general(g2, part, (((1,), (0,)), ((), ())),
                                       preferred_element_type=jnp.float32)

        def count_ge(cand):
            hit = keys_ref[...] >= cand
            return gsum(jnp.sum(jnp.where(hit, 1.0, 0.0), axis=1,
                                keepdims=True))

        n_nonneg = count_ge(jnp.zeros((rows, 1), jnp.int32))
        prefix = jnp.where(n_nonneg >= kf,
                           jnp.zeros((rows, 1), jnp.int32),
                           jnp.full((rows, 1), _INT_MIN, jnp.int32))

        def tbody(i, p):
            bit = jax.lax.shift_left(jnp.int32(1), jnp.int32(30) - i)
            cand = p | bit
            return jnp.where(count_ge(cand) >= kf, cand, p)

        thr = jax.lax.fori_loop(0, 31, tbody, prefix)

        keys = keys_ref[...]
        n_gt = gsum(jnp.sum(jnp.where(keys > thr, 1.0, 0.0), axis=1,
                            keepdims=True))
        n_ge = gsum(jnp.sum(jnp.where(keys >= thr, 1.0, 0.0), axis=1,
                            keepdims=True))
        r = kf - n_gt
        big = jnp.int32(1 << idx_bits)
        m0 = jnp.where(n_ge == kf, jnp.full((rows, 1), big),
                       jnp.zeros((rows, 1), jnp.int32))
        trips = jnp.where(jnp.any(n_ge != kf), idx_bits, 0)
        seg = jax.lax.rem(jax.lax.broadcasted_iota(jnp.int32, (rows, ll), 0),
                          jnp.int32(g))
        idx = seg * ll + jax.lax.broadcasted_iota(jnp.int32, (rows, ll), 1)

        def mbody(i, m):
            cand = m | jax.lax.shift_left(jnp.int32(1),
                                          jnp.int32(idx_bits - 1) - i)
            tied_below = (keys_ref[...] == thr) & (idx <= cand)
            cnt = gsum(jnp.sum(jnp.where(tied_below, 1.0, 0.0), axis=1,
                               keepdims=True))
            return jnp.where(cnt <= r, cand, m)

        m_fin = jax.lax.fori_loop(0, trips, mbody, m0)
        thr_ref[...] = thr.reshape(nc, g)[:, 0:1]
        m_ref[...] = m_fin.reshape(nc, g)[:, 0:1]

    kblk = _keys(scores_blk_ref[0])
    iblk = jax.lax.broadcasted_iota(jnp.int32, (nc, br), 1) + step * br
    thr = thr_ref[...]
    sel = (kblk > thr) | ((kblk == thr) & (iblk <= m_ref[...]))
    w = jnp.where(sel, 1.0, 0.0)
    out_ref[...] += jax.lax.dot_general(
        w, feat_ref[...], (((1,), (0,)), ((), ())),
        preferred_element_type=jnp.float32)

    @pl.when(step == nsteps - 1)
    def _finish():
        out_ref[...] *= scale_ref[0, 0]


def kernel(image_features, image_scores, xi_c):
    n, d = image_features.shape
    nc = image_scores.shape[1]
    k = n // nc
    br = 5000
    nsteps = n // br
    npad = ((n + 1023) // 1024) * 1024
    idx_bits = max(1, (npad - 1).bit_length())

    scores_t = image_scores.T
    scores_tp = jnp.pad(scores_t, ((0, 0), (0, npad - n)),
                        constant_values=-jnp.inf)
    scores_g = scores_tp.reshape(nc * 8, npad // 8)
    scores_blocks = scores_t.reshape(nc, nsteps, br).transpose(1, 0, 2)

    topk_mask = (jnp.arange(k) < k * xi_c).astype(image_features.dtype)
    scale = (1.0 / jnp.sum(topk_mask)).astype(jnp.float32).reshape(1, 1)

    body = functools.partial(_cc_kernel, nsteps=nsteps, br=br, k=k,
                             npad=npad, idx_bits=idx_bits)
    return pl.pallas_call(
        body,
        grid=(nsteps,),
        in_specs=[
            pl.BlockSpec((nc * 8, npad // 8), lambda j: (0, 0)),
            pl.BlockSpec((1, nc, br), lambda j: (j, 0, 0)),
            pl.BlockSpec((br, d), lambda j: (j, 0)),
            pl.BlockSpec(memory_space=pltpu.SMEM),
        ],
        out_specs=pl.BlockSpec((nc, d), lambda j: (0, 0)),
        out_shape=jax.ShapeDtypeStruct((nc, d), jnp.float32),
        scratch_shapes=[
            pltpu.VMEM((nc * 8, npad // 8), jnp.int32),
            pltpu.VMEM((nc, 1), jnp.int32),
            pltpu.VMEM((nc, 1), jnp.int32),
        ],
        compiler_params=pltpu.CompilerParams(
            dimension_semantics=("arbitrary",)),
    )(scores_g, scores_blocks, image_features, scale)
